# baseline (speedup 1.0000x reference)
import functools

import jax
import jax.numpy as jnp
import numpy as np
from jax import lax
from jax.experimental import pallas as pl
from jax.experimental.pallas import tpu as pltpu
from jax.experimental.pallas import tpu_sc as plsc

_PCM = np.array([
    [1, 0, 1, 0, 1, 0, 1, 0, 1, 0, 1, 0, 1, 0, 1, 0, 1, 0, 1, 0, 1, 0, 1, 0, 1, 0, 1, 0, 1, 0, 1],
    [0, 1, 1, 0, 0, 1, 1, 0, 0, 1, 1, 0, 0, 1, 1, 0, 0, 1, 1, 0, 0, 1, 1, 0, 0, 1, 1, 0, 0, 1, 1],
    [0, 0, 0, 1, 1, 1, 1, 0, 0, 0, 0, 1, 1, 1, 1, 0, 0, 0, 0, 1, 1, 1, 1, 0, 0, 0, 0, 1, 1, 1, 1],
    [0, 0, 0, 0, 0, 0, 0, 1, 1, 1, 1, 1, 1, 1, 1, 0, 0, 0, 0, 0, 0, 0, 0, 1, 1, 1, 1, 1, 1, 1, 1],
    [0, 0, 0, 0, 0, 0, 0, 0, 0, 0, 0, 0, 0, 0, 0, 1, 1, 1, 1, 1, 1, 1, 1, 1, 1, 1, 1, 1, 1, 1, 1],
], dtype=np.int64)
_ROLLED = np.stack(np.where(_PCM), axis=1)
_NCHK, _NVAR = _PCM.shape
_E = _ROLLED.shape[0]
_DEG = 16
_COLS = _ROLLED[:, 1].reshape(_NCHK, _DEG)
_COL_EDGES = [np.where(_ROLLED[:, 1] == v)[0].tolist() for v in range(_NVAR)]
_NUM_ITER = 20

_LN2 = 0.6931471805599453
_RLO = float(1e-7 / (2.0 - 1e-7))
_RHI = float((2.0 - 1e-7) / 1e-7)
_SQRT2 = 1.4142135


def _sc_log(r):
    i = plsc.bitcast(r, jnp.int32)
    k = (i >> 23) - 127
    m = plsc.bitcast((i & 0x7FFFFF) | 0x3F800000, jnp.float32)
    big = m > _SQRT2
    m = jnp.where(big, m * 0.5, m)
    kf = k.astype(jnp.float32) + jnp.where(big, 1.0, 0.0)
    z = (m - 1.0) / (m + 1.0)
    z2 = z * z
    p = z * (2.0 + z2 * (2.0 / 3.0 + z2 * (0.4 + z2 * (2.0 / 7.0 + z2 * (2.0 / 9.0)))))
    return kf * _LN2 + p


def _sc_tanh12(y):
    t = jnp.exp(-jnp.abs(y))
    q = (1.0 - t) / (1.0 + t)
    return jnp.where(y < 0.0, -q, q)


def _loo_products(grp):
    n = len(grp)
    pref = [grp[0]]
    for k in range(1, n):
        pref.append(pref[-1] * grp[k])
    suf = [grp[n - 1]]
    for k in range(n - 2, -1, -1):
        suf.append(suf[-1] * grp[k])
    suf = suf[::-1]
    out = []
    for k in range(n):
        if k == 0:
            out.append(suf[1])
        elif k == n - 1:
            out.append(pref[n - 2])
        else:
            out.append(pref[k - 1] * suf[k + 1])
    return out


def _sc_bp_body(pb, llr_hbm, out_hbm, llr_v, msg_v, he_v, out_v):
    wid = lax.axis_index("s") * 2 + lax.axis_index("c")
    pltpu.sync_copy(llr_hbm.at[wid], llr_v)

    def g_body(g, carry):
        lanes = pl.ds(g * 16, 16)
        for v in range(_NVAR):
            t = _sc_tanh12(llr_v[v, lanes])
            for e in _COL_EDGES[v]:
                msg_v[e, lanes] = t

        def it_body(it, c2):
            cs = [None] * _NVAR
            for c in range(_NCHK):
                grp = [msg_v[c * _DEG + k, lanes] for k in range(_DEG)]
                loo = _loo_products(grp)
                for k in range(_DEG):
                    r = (1.0 + loo[k]) / (1.0 - loo[k])
                    he = _sc_log(jnp.clip(r, _RLO, _RHI))
                    e = c * _DEG + k
                    he_v[e, lanes] = he
                    v = int(_COLS[c, k])
                    cs[v] = he if cs[v] is None else cs[v] + he
            for c in range(_NCHK):
                for k in range(_DEG):
                    e = c * _DEG + k
                    v = int(_COLS[c, k])
                    y = cs[v] - he_v[e, lanes] + llr_v[v, lanes]
                    msg_v[e, lanes] = _sc_tanh12(y)
            for v in range(_NVAR):
                out_v[v, lanes] = cs[v] + llr_v[v, lanes]
            return c2

        lax.fori_loop(0, _NUM_ITER, it_body, 0)
        return carry

    lax.fori_loop(0, pb // 16, g_body, 0)
    pltpu.sync_copy(out_v, out_hbm.at[wid])


_NW = 32


@functools.partial(jax.jit, static_argnames=("pb",))
def _sc_bp(llr_sc, pb):
    mesh = plsc.VectorSubcoreMesh(
        core_axis_name="c", subcore_axis_name="s", num_cores=2, num_subcores=16)
    return pl.kernel(
        functools.partial(_sc_bp_body, pb),
        out_type=jax.ShapeDtypeStruct((_NW, _NVAR, pb), jnp.float32),
        mesh=mesh,
        compiler_params=pltpu.CompilerParams(needs_layout_passes=False),
        scratch_types=[
            pltpu.VMEM((_NVAR, pb), jnp.float32),
            pltpu.VMEM((_E, pb), jnp.float32),
            pltpu.VMEM((_E, pb), jnp.float32),
            pltpu.VMEM((_NVAR, pb), jnp.float32),
        ],
    )(llr_sc)


@jax.jit
def kernel(llr):
    B = llr.shape[0]
    pb = B // _NW
    llr_sc = llr.T.reshape(_NVAR, _NW, pb).transpose(1, 0, 2)
    out_sc = _sc_bp(llr_sc, pb)
    return out_sc.transpose(1, 0, 2).reshape(_NVAR, B).T
